# Initial kernel scaffold; baseline (speedup 1.0000x reference)
import jax
import jax.numpy as jnp
from jax.experimental import pallas as pl


def kernel(x, emb):
    raise NotImplementedError("write your pallas kernel here")


# speedup vs baseline: 1.4847x; 1.4847x over previous
import jax
import jax.numpy as jnp
from jax.experimental import pallas as pl


_ROWS = 256


def _add_body(x_ref, e_ref, o_ref):
    o_ref[...] = x_ref[...] + e_ref[...][None, :, :]


def kernel(x, emb):
    B, S, D = x.shape
    grid = (S // _ROWS, B)
    return pl.pallas_call(
        _add_body,
        grid=grid,
        in_specs=[
            pl.BlockSpec((1, _ROWS, D), lambda i, b: (b, i, 0)),
            pl.BlockSpec((_ROWS, D), lambda i, b: (i, 0)),
        ],
        out_specs=pl.BlockSpec((1, _ROWS, D), lambda i, b: (b, i, 0)),
        out_shape=jax.ShapeDtypeStruct((B, S, D), x.dtype),
    )(x, emb)
